# Initial kernel scaffold; baseline (speedup 1.0000x reference)
import jax
import jax.numpy as jnp
from jax.experimental import pallas as pl


def kernel(x, edge_index, edge_weight, W, b):
    raise NotImplementedError("write your pallas kernel here")


# speedup vs baseline: 6.3706x; 6.3706x over previous
import functools

import jax
import jax.numpy as jnp
from jax import lax
from jax.experimental import pallas as pl
from jax.experimental.pallas import tpu as pltpu
from jax.experimental.pallas import tpu_sc as plsc

N = 10000
E = 320000
D = 128

NC = 2
NS = 16
NW = NC * NS

CHUNK = 128
NCHUNK = 80
EPW = NCHUNK * CHUNK
EPAD = NW * EPW
SBLK = 16
ROWS_PT = 624
TAIL = N - NS * ROWS_PT
ZR = 16


def _tc_linear(x, W, b):
    blk = 1000

    def body(x_ref, w_ref, b_ref, o_ref):
        o_ref[...] = (
            lax.dot_general(
                x_ref[...], w_ref[...],
                (((1,), (1,)), ((), ())),
                preferred_element_type=jnp.float32,
            )
            + b_ref[...]
        )

    return pl.pallas_call(
        body,
        grid=(N // blk,),
        in_specs=[
            pl.BlockSpec((blk, D), lambda i: (i, 0)),
            pl.BlockSpec((D, D), lambda i: (0, 0)),
            pl.BlockSpec((1, D), lambda i: (0, 0)),
        ],
        out_specs=pl.BlockSpec((blk, D), lambda i: (i, 0)),
        out_shape=jax.ShapeDtypeStruct((N, D), jnp.float32),
    )(x, W, b.reshape(1, D))


def _tc_combine(partials):
    blk = 1000

    def body(p_ref, o_ref):
        o_ref[...] = p_ref[0] + p_ref[1]

    return pl.pallas_call(
        body,
        grid=(N // blk,),
        in_specs=[pl.BlockSpec((2, blk, D), lambda i: (0, i, 0))],
        out_specs=pl.BlockSpec((blk, D), lambda i: (i, 0)),
        out_shape=jax.ShapeDtypeStruct((N, D), jnp.float32),
    )(partials)


def _sc_spmm(z, src3, dst3, w3):
    mesh = plsc.VectorSubcoreMesh(core_axis_name="c", subcore_axis_name="s")

    @functools.partial(
        pl.kernel,
        out_type=jax.ShapeDtypeStruct((NC, N, D), jnp.float32),
        mesh=mesh,
        scratch_types=[
            pltpu.VMEM_SHARED((N, D), jnp.float32),
            pltpu.VMEM((SBLK, CHUNK), jnp.int32),
            pltpu.VMEM((SBLK, CHUNK), jnp.int32),
            pltpu.VMEM((SBLK, CHUNK), jnp.float32),
            pltpu.VMEM((CHUNK, D), jnp.float32),
            pltpu.VMEM((ZR, D), jnp.float32),
            pltpu.SemaphoreType.DMA,
        ],
    )
    def spmm(z_hbm, src_hbm, dst_hbm, w_hbm, out_hbm,
             acc_sh, src_v, dst_v, w_v, rows_v, zero_v, sem):
        cid = lax.axis_index("c")
        sid = lax.axis_index("s")
        wid = cid * NS + sid

        def zrow(r, _):
            for q in range(D // 16):
                zero_v[r, pl.ds(q * 16, 16)] = jnp.zeros((16,), jnp.float32)
            return 0
        lax.fori_loop(0, ZR, zrow, 0)

        def zcopy(j, _):
            pltpu.sync_copy(zero_v, acc_sh.at[pl.ds(sid * ROWS_PT + j * ZR, ZR)])
            return 0
        lax.fori_loop(0, ROWS_PT // ZR, zcopy, 0)

        @pl.when(sid == NS - 1)
        def _():
            pltpu.sync_copy(zero_v.at[pl.ds(0, TAIL)],
                            acc_sh.at[pl.ds(NS * ROWS_PT, TAIL)])
        plsc.subcore_barrier()

        def sblock(s, _):
            bsl = pl.ds(s * SBLK, SBLK)
            pltpu.sync_copy(src_hbm.at[wid, bsl], src_v)
            pltpu.sync_copy(dst_hbm.at[wid, bsl], dst_v)
            pltpu.sync_copy(w_hbm.at[wid, bsl], w_v)

            def chunk(k, _):
                pltpu.async_copy(z_hbm.at[src_v.at[k]], rows_v, sem).wait()

                def egroup(g, _):
                    wv = w_v[k, pl.ds(g * 16, 16)]
                    for j in range(16):
                        e = g * 16 + j
                        wt = wv[j]
                        for q in range(D // 16):
                            sl = pl.ds(q * 16, 16)
                            rows_v[e, sl] = rows_v[e, sl] * wt
                    return 0
                lax.fori_loop(0, CHUNK // 16, egroup, 0)

                pltpu.sync_copy(rows_v, acc_sh.at[dst_v.at[k]], add=True)
                return 0
            lax.fori_loop(0, SBLK, chunk, 0)
            return 0
        lax.fori_loop(0, NCHUNK // SBLK, sblock, 0)

        plsc.subcore_barrier()
        sl = pl.ds(sid * ROWS_PT, ROWS_PT)
        pltpu.sync_copy(acc_sh.at[sl], out_hbm.at[cid, sl])

        @pl.when(sid == NS - 1)
        def _():
            tl = pl.ds(NS * ROWS_PT, TAIL)
            pltpu.sync_copy(acc_sh.at[tl], out_hbm.at[cid, tl])

    return spmm(z, src3, dst3, w3)


def kernel(x, edge_index, edge_weight, W, b):
    pad = EPAD - E
    pad_idx = jnp.arange(pad, dtype=jnp.int32) % N
    src3 = jnp.concatenate([edge_index[0], pad_idx]).reshape(NW, NCHUNK, CHUNK)
    dst3 = jnp.concatenate([edge_index[1], pad_idx]).reshape(NW, NCHUNK, CHUNK)
    w3 = jnp.concatenate(
        [edge_weight, jnp.zeros((pad,), jnp.float32)]).reshape(NW, NCHUNK, CHUNK)

    z = _tc_linear(x, W, b)
    for _ in range(2):
        partials = _sc_spmm(z, src3, dst3, w3)
        z = _tc_combine(partials)
    return z


# speedup vs baseline: 9.1569x; 1.4374x over previous
import functools

import jax
import jax.numpy as jnp
from jax import lax
from jax.experimental import pallas as pl
from jax.experimental.pallas import tpu as pltpu
from jax.experimental.pallas import tpu_sc as plsc

N = 10000
E = 320000
D = 128

NC = 2
NS = 16
NW = NC * NS

CHUNK = 128
NCHUNK = 80
EPW = NCHUNK * CHUNK
EPAD = NW * EPW
IBLK = 40
NPAIR = IBLK // 2
ROWS_PT = 624
TAIL = N - NS * ROWS_PT
ZR = 16


def _tc_linear(x, W, b):
    blk = 1000

    def body(x_ref, w_ref, b_ref, o_ref):
        o_ref[...] = (
            lax.dot_general(
                x_ref[...], w_ref[...],
                (((1,), (1,)), ((), ())),
                preferred_element_type=jnp.float32,
            )
            + b_ref[...]
        )

    return pl.pallas_call(
        body,
        grid=(N // blk,),
        in_specs=[
            pl.BlockSpec((blk, D), lambda i: (i, 0)),
            pl.BlockSpec((D, D), lambda i: (0, 0)),
            pl.BlockSpec((1, D), lambda i: (0, 0)),
        ],
        out_specs=pl.BlockSpec((blk, D), lambda i: (i, 0)),
        out_shape=jax.ShapeDtypeStruct((N, D), jnp.float32),
    )(x, W, b.reshape(1, D))


def _tc_combine(partials):
    blk = 1000

    def body(p_ref, o_ref):
        o_ref[...] = p_ref[0] + p_ref[1]

    return pl.pallas_call(
        body,
        grid=(N // blk,),
        in_specs=[pl.BlockSpec((2, blk, D), lambda i: (0, i, 0))],
        out_specs=pl.BlockSpec((blk, D), lambda i: (i, 0)),
        out_shape=jax.ShapeDtypeStruct((N, D), jnp.float32),
    )(partials)


def _sc_spmm(z, src3, dst3, w3):
    mesh = plsc.VectorSubcoreMesh(core_axis_name="c", subcore_axis_name="s")

    @functools.partial(
        pl.kernel,
        out_type=jax.ShapeDtypeStruct((NC, N, D), jnp.float32),
        mesh=mesh,
        scratch_types=[
            pltpu.VMEM_SHARED((N, D), jnp.float32),
            pltpu.VMEM((IBLK, CHUNK), jnp.int32),
            pltpu.VMEM((IBLK, CHUNK), jnp.int32),
            pltpu.VMEM((IBLK, CHUNK), jnp.float32),
            pltpu.VMEM((CHUNK, D), jnp.float32),
            pltpu.VMEM((CHUNK, D), jnp.float32),
            pltpu.SemaphoreType.DMA,
            pltpu.SemaphoreType.DMA,
            pltpu.SemaphoreType.DMA,
            pltpu.SemaphoreType.DMA,
        ],
    )
    def spmm(z_hbm, src_hbm, dst_hbm, w_hbm, out_hbm,
             acc_sh, src_v, dst_v, w_v, rows0, rows1,
             gsem0, gsem1, ssem0, ssem1):
        cid = lax.axis_index("c")
        sid = lax.axis_index("s")
        wid = cid * NS + sid

        def zrow(r, _):
            for q in range(D // 16):
                rows0[r, pl.ds(q * 16, 16)] = jnp.zeros((16,), jnp.float32)
            return 0
        lax.fori_loop(0, CHUNK, zrow, 0)
        for j in range(ROWS_PT // CHUNK):
            pltpu.sync_copy(rows0,
                            acc_sh.at[pl.ds(sid * ROWS_PT + j * CHUNK, CHUNK)])
        rem = ROWS_PT % CHUNK
        pltpu.sync_copy(
            rows0.at[pl.ds(0, rem)],
            acc_sh.at[pl.ds(sid * ROWS_PT + (ROWS_PT // CHUNK) * CHUNK, rem)])

        @pl.when(sid == NS - 1)
        def _():
            pltpu.sync_copy(rows0.at[pl.ds(0, TAIL)],
                            acc_sh.at[pl.ds(NS * ROWS_PT, TAIL)])
        plsc.subcore_barrier()

        def mult(rv, k):
            def egroup(g, _):
                wv = w_v[k, pl.ds(g * 16, 16)]
                for j in range(16):
                    e = g * 16 + j
                    wt = wv[j]
                    for q in range(D // 16):
                        sl = pl.ds(q * 16, 16)
                        rv[e, sl] = rv[e, sl] * wt
                return 0
            lax.fori_loop(0, CHUNK // 16, egroup, 0)

        for s in range(NCHUNK // IBLK):
            bsl = pl.ds(s * IBLK, IBLK)
            pltpu.sync_copy(src_hbm.at[wid, bsl], src_v)
            pltpu.sync_copy(dst_hbm.at[wid, bsl], dst_v)
            pltpu.sync_copy(w_hbm.at[wid, bsl], w_v)

            pltpu.async_copy(z_hbm.at[src_v.at[0]], rows0, gsem0)
            pltpu.async_copy(z_hbm.at[src_v.at[1]], rows1, gsem1)

            def pair(i, _):
                k0 = 2 * i
                k1 = 2 * i + 1
                pltpu.make_async_copy(z_hbm.at[src_v.at[k0]], rows0, gsem0).wait()
                mult(rows0, k0)
                pltpu.async_copy(rows0, acc_sh.at[dst_v.at[k0]], ssem0, add=True)

                pltpu.make_async_copy(z_hbm.at[src_v.at[k1]], rows1, gsem1).wait()
                mult(rows1, k1)
                pltpu.async_copy(rows1, acc_sh.at[dst_v.at[k1]], ssem1, add=True)

                @pl.when(i < NPAIR - 1)
                def _():
                    pltpu.make_async_copy(
                        rows0, acc_sh.at[dst_v.at[k0]], ssem0).wait()
                    pltpu.async_copy(z_hbm.at[src_v.at[k0 + 2]], rows0, gsem0)
                    pltpu.make_async_copy(
                        rows1, acc_sh.at[dst_v.at[k1]], ssem1).wait()
                    pltpu.async_copy(z_hbm.at[src_v.at[k1 + 2]], rows1, gsem1)

                @pl.when(i == NPAIR - 1)
                def _():
                    pltpu.make_async_copy(
                        rows0, acc_sh.at[dst_v.at[k0]], ssem0).wait()
                    pltpu.make_async_copy(
                        rows1, acc_sh.at[dst_v.at[k1]], ssem1).wait()
                return 0
            lax.fori_loop(0, NPAIR, pair, 0)

        plsc.subcore_barrier()
        sl = pl.ds(sid * ROWS_PT, ROWS_PT)
        pltpu.sync_copy(acc_sh.at[sl], out_hbm.at[cid, sl])

        @pl.when(sid == NS - 1)
        def _():
            tl = pl.ds(NS * ROWS_PT, TAIL)
            pltpu.sync_copy(acc_sh.at[tl], out_hbm.at[cid, tl])

    return spmm(z, src3, dst3, w3)


def kernel(x, edge_index, edge_weight, W, b):
    pad = EPAD - E
    pad_idx = jnp.arange(pad, dtype=jnp.int32) % N
    src3 = jnp.concatenate([edge_index[0], pad_idx]).reshape(NW, NCHUNK, CHUNK)
    dst3 = jnp.concatenate([edge_index[1], pad_idx]).reshape(NW, NCHUNK, CHUNK)
    w3 = jnp.concatenate(
        [edge_weight, jnp.zeros((pad,), jnp.float32)]).reshape(NW, NCHUNK, CHUNK)

    z = _tc_linear(x, W, b)
    for _ in range(2):
        partials = _sc_spmm(z, src3, dst3, w3)
        z = _tc_combine(partials)
    return z


# speedup vs baseline: 9.4708x; 1.0343x over previous
import functools

import jax
import jax.numpy as jnp
from jax import lax
from jax.experimental import pallas as pl
from jax.experimental.pallas import tpu as pltpu
from jax.experimental.pallas import tpu_sc as plsc

N = 10000
E = 320000
D = 128

NC = 2
NS = 16
NW = NC * NS

CHUNK = 80
NCHUNK = 128
EPW = NCHUNK * CHUNK
EPAD = NW * EPW
IBLK = 16
NBUF = 4
ROWS_PT = 624
TAIL = N - NS * ROWS_PT
ZR = 16


def _tc_linear(x, W, b):
    blk = 1000

    def body(x_ref, w_ref, b_ref, o_ref):
        o_ref[...] = (
            lax.dot_general(
                x_ref[...], w_ref[...],
                (((1,), (1,)), ((), ())),
                preferred_element_type=jnp.float32,
            )
            + b_ref[...]
        )

    return pl.pallas_call(
        body,
        grid=(N // blk,),
        in_specs=[
            pl.BlockSpec((blk, D), lambda i: (i, 0)),
            pl.BlockSpec((D, D), lambda i: (0, 0)),
            pl.BlockSpec((1, D), lambda i: (0, 0)),
        ],
        out_specs=pl.BlockSpec((blk, D), lambda i: (i, 0)),
        out_shape=jax.ShapeDtypeStruct((N, D), jnp.float32),
    )(x, W, b.reshape(1, D))


def _tc_combine(partials):
    blk = 1000

    def body(p_ref, o_ref):
        o_ref[...] = p_ref[0] + p_ref[1]

    return pl.pallas_call(
        body,
        grid=(N // blk,),
        in_specs=[pl.BlockSpec((2, blk, D), lambda i: (0, i, 0))],
        out_specs=pl.BlockSpec((blk, D), lambda i: (i, 0)),
        out_shape=jax.ShapeDtypeStruct((N, D), jnp.float32),
    )(partials)


def _sc_spmm(z, src3, dst3, w3):
    mesh = plsc.VectorSubcoreMesh(core_axis_name="c", subcore_axis_name="s")

    @functools.partial(
        pl.kernel,
        out_type=jax.ShapeDtypeStruct((NC, N, D), jnp.float32),
        mesh=mesh,
        scratch_types=[
            pltpu.VMEM_SHARED((N, D), jnp.float32),
            pltpu.VMEM((IBLK, CHUNK), jnp.int32),
            pltpu.VMEM((IBLK, CHUNK), jnp.int32),
            pltpu.VMEM((IBLK, CHUNK), jnp.float32),
        ]
        + [pltpu.VMEM((CHUNK, D), jnp.float32)] * NBUF
        + [pltpu.SemaphoreType.DMA] * (2 * NBUF),
    )
    def spmm(z_hbm, src_hbm, dst_hbm, w_hbm, out_hbm,
             acc_sh, src_v, dst_v, w_v, *bufs_and_sems):
        rows = list(bufs_and_sems[:NBUF])
        gsem = list(bufs_and_sems[NBUF:2 * NBUF])
        ssem = list(bufs_and_sems[2 * NBUF:])
        cid = lax.axis_index("c")
        sid = lax.axis_index("s")
        wid = cid * NS + sid

        def zrow(r, _):
            for q in range(D // 16):
                rows[0][r, pl.ds(q * 16, 16)] = jnp.zeros((16,), jnp.float32)
            return 0
        lax.fori_loop(0, CHUNK, zrow, 0)
        for j in range(ROWS_PT // CHUNK):
            pltpu.sync_copy(rows[0],
                            acc_sh.at[pl.ds(sid * ROWS_PT + j * CHUNK, CHUNK)])
        rem = ROWS_PT % CHUNK
        pltpu.sync_copy(
            rows[0].at[pl.ds(0, rem)],
            acc_sh.at[pl.ds(sid * ROWS_PT + (ROWS_PT // CHUNK) * CHUNK, rem)])

        @pl.when(sid == NS - 1)
        def _():
            pltpu.sync_copy(rows[0].at[pl.ds(0, TAIL)],
                            acc_sh.at[pl.ds(NS * ROWS_PT, TAIL)])
        plsc.subcore_barrier()

        def mult(rv, k):
            def egroup(g, _):
                wv = w_v[k, pl.ds(g * 16, 16)]
                for j in range(16):
                    e = g * 16 + j
                    wt = wv[j]
                    for q in range(D // 16):
                        sl = pl.ds(q * 16, 16)
                        rv[e, sl] = rv[e, sl] * wt
                return 0
            lax.fori_loop(0, CHUNK // 16, egroup, 0)

        def gather(k, b):
            return pltpu.async_copy(z_hbm.at[src_v.at[k]], rows[b], gsem[b])

        def gather_wait(k, b):
            pltpu.make_async_copy(z_hbm.at[src_v.at[k]], rows[b], gsem[b]).wait()

        def scatter(k, b):
            return pltpu.async_copy(rows[b], acc_sh.at[dst_v.at[k]], ssem[b],
                                    add=True)

        def scatter_wait(k, b):
            pltpu.make_async_copy(rows[b], acc_sh.at[dst_v.at[k]],
                                  ssem[b]).wait()

        def sblock(s, _):
            bsl = pl.ds(s * IBLK, IBLK)
            pltpu.sync_copy(src_hbm.at[wid, bsl], src_v)
            pltpu.sync_copy(dst_hbm.at[wid, bsl], dst_v)
            pltpu.sync_copy(w_hbm.at[wid, bsl], w_v)

            gather(0, 0)
            gather(1, 1)

            def quad(q, _):
                for j in range(NBUF):
                    kk = q * NBUF + j
                    j2 = (j + 2) % NBUF
                    gather_wait(kk, j)
                    mult(rows[j], kk)
                    scatter(kk, j)

                    @pl.when(jnp.logical_and(kk >= 2, kk <= IBLK - 3))
                    def _():
                        scatter_wait(kk - 2, j2)
                        gather(kk + 2, j2)

                    @pl.when(kk < 2)
                    def _():
                        gather(kk + 2, j2)
                return 0
            lax.fori_loop(0, IBLK // NBUF, quad, 0)

            for j in range(NBUF):
                scatter_wait(IBLK - NBUF + j, (IBLK - NBUF + j) % NBUF)
            return 0
        lax.fori_loop(0, NCHUNK // IBLK, sblock, 0)

        plsc.subcore_barrier()
        sl = pl.ds(sid * ROWS_PT, ROWS_PT)
        pltpu.sync_copy(acc_sh.at[sl], out_hbm.at[cid, sl])

        @pl.when(sid == NS - 1)
        def _():
            tl = pl.ds(NS * ROWS_PT, TAIL)
            pltpu.sync_copy(acc_sh.at[tl], out_hbm.at[cid, tl])

    return spmm(z, src3, dst3, w3)


def kernel(x, edge_index, edge_weight, W, b):
    pad = EPAD - E
    pad_idx = jnp.arange(pad, dtype=jnp.int32) % N
    src3 = jnp.concatenate([edge_index[0], pad_idx]).reshape(NW, NCHUNK, CHUNK)
    dst3 = jnp.concatenate([edge_index[1], pad_idx]).reshape(NW, NCHUNK, CHUNK)
    w3 = jnp.concatenate(
        [edge_weight, jnp.zeros((pad,), jnp.float32)]).reshape(NW, NCHUNK, CHUNK)

    z = _tc_linear(x, W, b)
    for _ in range(2):
        partials = _sc_spmm(z, src3, dst3, w3)
        z = _tc_combine(partials)
    return z
